# Initial kernel scaffold; baseline (speedup 1.0000x reference)
import functools

import jax
import jax.numpy as jnp
from jax import lax
from jax.experimental import pallas as pl
from jax.experimental.pallas import tpu as pltpu
from jax.experimental.pallas import tpu_sc as plsc

N = 100000
E = 1600000
NP = 102400
CH = 25600
CHP = CH + 16
NC = 2
NT = 16
L = 16

_MESH = dict(core_axis_name="c", subcore_axis_name="s")


def _sds(shape, dtype=jnp.float32):
    return jax.ShapeDtypeStruct(shape, dtype)


_B1 = 10000
_EW1 = E // (NC * NT)


def _deg_body(ei_hbm, ones_hbm, zeros_hbm, out_hbm, acc, idx, ones_v, sem):
    c = lax.axis_index("c")
    s = lax.axis_index("s")
    wid = c * NT + s
    zs = s * (NP // NT)
    pltpu.sync_copy(zeros_hbm.at[pl.ds(zs, NP // NT)], acc.at[pl.ds(zs, NP // NT)])
    pltpu.sync_copy(ones_hbm, ones_v)
    plsc.subcore_barrier()
    base = wid * _EW1
    for b in range(_EW1 // _B1):
        pltpu.sync_copy(ei_hbm.at[1, pl.ds(base + b * _B1, _B1)], idx)
        pltpu.sync_copy(ones_v, acc.at[idx], add=True)
    plsc.subcore_barrier()
    pltpu.sync_copy(acc.at[pl.ds(zs, NP // NT)],
                    out_hbm.at[c, pl.ds(zs, NP // NT)])


_deg_kernel = functools.partial(
    pl.kernel,
    out_type=_sds((NC, NP, 1)),
    mesh=plsc.VectorSubcoreMesh(**_MESH),
    scratch_types=[
        pltpu.VMEM_SHARED((NP, 1), jnp.float32),
        pltpu.VMEM((_B1,), jnp.int32),
        pltpu.VMEM((_B1, 1), jnp.float32),
        pltpu.SemaphoreType.DMA,
    ],
)(_deg_body)


_B2 = 2000
_EW2 = E // (NC * NT)


def _agg2_body(ei_hbm, g0_hbm, zeros_hbm, out_hbm, acc, srcb, dstb, rows, sem):
    c = lax.axis_index("c")
    s = lax.axis_index("s")
    wid = c * NT + s
    zs = s * (NP // NT)
    pltpu.sync_copy(zeros_hbm.at[pl.ds(zs, NP // NT), :], acc.at[pl.ds(zs, NP // NT), :])
    plsc.subcore_barrier()
    base = wid * _EW2
    for b in range(_EW2 // _B2):
        off = base + b * _B2
        pltpu.sync_copy(ei_hbm.at[0, pl.ds(off, _B2)], srcb)
        pltpu.sync_copy(ei_hbm.at[1, pl.ds(off, _B2)], dstb)
        pltpu.async_copy(g0_hbm.at[srcb], rows, sem).wait()
        pltpu.sync_copy(rows, acc.at[dstb], add=True)
    plsc.subcore_barrier()
    pltpu.sync_copy(acc.at[pl.ds(zs, NP // NT), :],
                    out_hbm.at[c, pl.ds(zs, NP // NT), :])


_agg2_kernel = functools.partial(
    pl.kernel,
    out_type=_sds((NC, NP, 2)),
    mesh=plsc.VectorSubcoreMesh(**_MESH),
    scratch_types=[
        pltpu.VMEM_SHARED((NP, 2), jnp.float32),
        pltpu.VMEM((_B2,), jnp.int32),
        pltpu.VMEM((_B2,), jnp.int32),
        pltpu.VMEM((_B2, 2), jnp.float32),
        pltpu.SemaphoreType.DMA,
    ],
)(_agg2_body)


_B3 = 800
_G = 1024
_ET = E // NT
_NB3 = _ET // _B3
_CPT = CHP // NT


def _agg64_body(ei_hbm, g1_hbm, zeros_hbm, out_hbm,
                acc, srcb, dstb, csrc, cldst, fsrc, fldst, rows, sem):
    c = lax.axis_index("c")
    s = lax.axis_index("s")
    lanes = jnp.arange(L, dtype=jnp.int32)

    def fire():
        def cp(i, _):
            fsrc[pl.ds(i * L, L)] = csrc[pl.ds(i * L, L)]
            fldst[pl.ds(i * L, L)] = cldst[pl.ds(i * L, L)]
            return 0
        lax.fori_loop(0, _G // L, cp, 0)
        pltpu.async_copy(g1_hbm.at[fsrc], rows, sem).wait()
        pltpu.sync_copy(rows, acc.at[fldst], add=True)

    for jx in range(2):
        job = c + 2 * jx
        base = job * CH
        zs = s * _CPT
        pltpu.sync_copy(zeros_hbm.at[pl.ds(zs, _CPT), :], acc.at[pl.ds(zs, _CPT), :])
        plsc.subcore_barrier()

        ebase = s * _ET

        def batch_step(b, cnt):
            off = ebase + b * _B3
            pltpu.sync_copy(ei_hbm.at[0, pl.ds(off, _B3)], srcb)
            pltpu.sync_copy(ei_hbm.at[1, pl.ds(off, _B3)], dstb)

            def filt(i, cnt):
                dstv = dstb[pl.ds(i * L, L)]
                srcv = srcb[pl.ds(i * L, L)]
                loc = dstv - base
                msk = (loc >= 0) & (loc < CH)
                inc = plsc.cumsum(jnp.where(msk, 1, 0).astype(jnp.int32))
                pos = cnt + inc - 1
                plsc.store_scatter(csrc, [pos], srcv, mask=msk)
                plsc.store_scatter(cldst, [pos], jnp.where(msk, loc, 0), mask=msk)
                return cnt + jnp.sum(jnp.where(msk, 1, 0).astype(jnp.int32))

            cnt = lax.fori_loop(0, _B3 // L, filt, cnt)

            def do_fire(cnt):
                fire()
                def sh(i, _):
                    csrc[pl.ds(i * L, L)] = csrc[pl.ds(_G + i * L, L)]
                    cldst[pl.ds(i * L, L)] = cldst[pl.ds(_G + i * L, L)]
                    return 0
                lax.fori_loop(0, _B3 // L, sh, 0)
                return cnt - _G

            return lax.cond(cnt >= _G, do_fire, lambda cnt: cnt, cnt)

        cnt = lax.fori_loop(0, _NB3, batch_step, jnp.int32(0))

        def pad(i, cnt):
            p = i * L + lanes
            msk = p >= cnt
            plsc.store_scatter(csrc, [p], p + s * _G, mask=msk)
            plsc.store_scatter(cldst, [p], CH + lanes, mask=msk)
            return cnt
        lax.fori_loop(0, _G // L, pad, cnt)
        fire()

        plsc.subcore_barrier()
        ws = s * (CH // NT)
        pltpu.sync_copy(acc.at[pl.ds(ws, CH // NT), :],
                        out_hbm.at[pl.ds(base + ws, CH // NT), :])
        plsc.subcore_barrier()


_agg64_kernel = functools.partial(
    pl.kernel,
    out_type=_sds((NP, 64)),
    mesh=plsc.VectorSubcoreMesh(**_MESH),
    scratch_types=[
        pltpu.VMEM_SHARED((CHP, 64), jnp.float32),
        pltpu.VMEM((_B3,), jnp.int32),
        pltpu.VMEM((_B3,), jnp.int32),
        pltpu.VMEM((2048,), jnp.int32),
        pltpu.VMEM((2048,), jnp.int32),
        pltpu.VMEM((_G,), jnp.int32),
        pltpu.VMEM((_G,), jnp.int32),
        pltpu.VMEM((_G, 64), jnp.float32),
        pltpu.SemaphoreType.DMA,
    ],
)(_agg64_body)


_BN = 2048


def _tc1_body(deg2_ref, x_ref, dinv_ref, g0_ref):
    deg = deg2_ref[0, :, 0] + deg2_ref[1, :, 0] + 1.0
    dinv = lax.rsqrt(deg)
    dinv_ref[...] = dinv
    g0_ref[...] = x_ref[...] * dinv[:, None]


def _tc1(deg2, x_pad):
    return pl.pallas_call(
        _tc1_body,
        grid=(NP // _BN,),
        in_specs=[
            pl.BlockSpec((NC, _BN, 1), lambda i: (0, i, 0)),
            pl.BlockSpec((_BN, 2), lambda i: (i, 0)),
        ],
        out_specs=[
            pl.BlockSpec((_BN,), lambda i: (i,)),
            pl.BlockSpec((_BN, 2), lambda i: (i, 0)),
        ],
        out_shape=[_sds((NP,)), _sds((NP, 2))],
    )(deg2, x_pad)


def _tc2_body(ag_ref, g0_ref, dinv_ref, w1_ref, b1_ref, g1_ref):
    dinv = dinv_ref[...]
    z = dinv[:, None] * (ag_ref[0] + ag_ref[1] + g0_ref[...])
    h1 = z[:, 0:1] * w1_ref[0, :][None, :] + z[:, 1:2] * w1_ref[1, :][None, :]
    h1 = jnp.maximum(h1 + b1_ref[...][None, :], 0.0)
    g1_ref[...] = h1 * dinv[:, None]


def _tc2(agg0, g0, dinv, W1, b1):
    return pl.pallas_call(
        _tc2_body,
        grid=(NP // _BN,),
        in_specs=[
            pl.BlockSpec((NC, _BN, 2), lambda i: (0, i, 0)),
            pl.BlockSpec((_BN, 2), lambda i: (i, 0)),
            pl.BlockSpec((_BN,), lambda i: (i,)),
            pl.BlockSpec((2, 64), lambda i: (0, 0)),
            pl.BlockSpec((64,), lambda i: (0,)),
        ],
        out_specs=pl.BlockSpec((_BN, 64), lambda i: (i, 0)),
        out_shape=_sds((NP, 64)),
    )(agg0, g0, dinv, W1, b1)


def _tc3_body(ag_ref, g1_ref, dinv_ref, w2_ref, b2_ref, wfc_ref, bfc_ref, o_ref):
    dinv = dinv_ref[...]
    z = dinv[:, None] * (ag_ref[...] + g1_ref[...])
    h2 = jnp.dot(z, w2_ref[...], preferred_element_type=jnp.float32)
    h2 = jnp.maximum(h2 + b2_ref[...][None, :], 0.0)
    o = jnp.dot(h2, wfc_ref[...], preferred_element_type=jnp.float32)
    o = o[:, 0] + bfc_ref[0]
    o_ref[...] = 1.0 / (1.0 + jnp.exp(-o))


def _tc3(agg1, g1, dinv, W2, b2, Wfc, bfc):
    return pl.pallas_call(
        _tc3_body,
        grid=(NP // _BN,),
        in_specs=[
            pl.BlockSpec((_BN, 64), lambda i: (i, 0)),
            pl.BlockSpec((_BN, 64), lambda i: (i, 0)),
            pl.BlockSpec((_BN,), lambda i: (i,)),
            pl.BlockSpec((64, 64), lambda i: (0, 0)),
            pl.BlockSpec((64,), lambda i: (0,)),
            pl.BlockSpec((64, 1), lambda i: (0, 0)),
            pl.BlockSpec((1,), lambda i: (0,)),
        ],
        out_specs=pl.BlockSpec((_BN,), lambda i: (i,)),
        out_shape=_sds((NP,)),
    )(agg1, g1, dinv, W2, b2, Wfc, bfc)


def kernel(x, edge_index, W1, b1, W2, b2, Wfc, bfc):
    ei = edge_index.astype(jnp.int32)
    x_pad = jnp.pad(x, ((0, NP - N), (0, 0)))
    zeros1 = jnp.zeros((NP, 1), jnp.float32)
    ones1 = jnp.ones((_B1, 1), jnp.float32)
    zeros2 = jnp.zeros((NP, 2), jnp.float32)
    zeros64 = jnp.zeros((CHP, 64), jnp.float32)

    deg2 = _deg_kernel(ei, ones1, zeros1)
    dinv, g0 = _tc1(deg2, x_pad)
    agg0 = _agg2_kernel(ei, g0, zeros2)
    g1 = _tc2(agg0, g0, dinv, W1, b1)
    agg1 = _agg64_kernel(ei, g1, zeros64)
    o = _tc3(agg1, g1, dinv, W2, b2, Wfc, bfc)
    return o[:N]


# speedup vs baseline: 24.2846x; 24.2846x over previous
import functools

import jax
import jax.numpy as jnp
from jax import lax
from jax.experimental import pallas as pl
from jax.experimental.pallas import tpu as pltpu
from jax.experimental.pallas import tpu_sc as plsc

N = 100000
E = 1600000
NP = 102400
CH = 25600
CHP = CH + 16
NC = 2
NT = 16
L = 16

_MESH = dict(core_axis_name="c", subcore_axis_name="s")


def _sds(shape, dtype=jnp.float32):
    return jax.ShapeDtypeStruct(shape, dtype)


_B1 = 1000
_EW1 = E // (NC * NT)


def _deg_body(dst_hbm, ones_hbm, zeros_hbm, out_hbm, acc, idx, ones_v, sem):
    c = lax.axis_index("c")
    s = lax.axis_index("s")
    wid = c * NT + s
    zs = s * (NP // NT)
    pltpu.sync_copy(zeros_hbm.at[pl.ds(zs, NP // NT), :], acc.at[pl.ds(zs, NP // NT), :])
    pltpu.sync_copy(ones_hbm, ones_v)
    plsc.subcore_barrier()
    base = wid * _EW1
    for b in range(_EW1 // _B1):
        pltpu.sync_copy(dst_hbm.at[pl.ds(base + b * _B1, _B1)], idx)
        pltpu.sync_copy(ones_v, acc.at[idx], add=True)
    plsc.subcore_barrier()
    pltpu.sync_copy(acc.at[pl.ds(zs, NP // NT), :],
                    out_hbm.at[c, pl.ds(zs, NP // NT), :])


_deg_kernel = functools.partial(
    pl.kernel,
    out_type=_sds((NC, NP, 16)),
    mesh=plsc.VectorSubcoreMesh(**_MESH),
    compiler_params=pltpu.CompilerParams(use_tc_tiling_on_sc=False, needs_layout_passes=False),
    scratch_types=[
        pltpu.VMEM_SHARED((NP, 16), jnp.float32),
        pltpu.VMEM((_B1,), jnp.int32),
        pltpu.VMEM((_B1, 16), jnp.float32),
        pltpu.SemaphoreType.DMA,
    ],
)(_deg_body)


_B2 = 1000
_EW2 = E // (NC * NT)


def _agg2_body(src_hbm, dst_hbm, g0_hbm, zeros_hbm, out_hbm, acc, srcb, dstb, rows, sem):
    c = lax.axis_index("c")
    s = lax.axis_index("s")
    wid = c * NT + s
    zs = s * (NP // NT)
    pltpu.sync_copy(zeros_hbm.at[pl.ds(zs, NP // NT), :], acc.at[pl.ds(zs, NP // NT), :])
    plsc.subcore_barrier()
    base = wid * _EW2
    for b in range(_EW2 // _B2):
        off = base + b * _B2
        pltpu.sync_copy(src_hbm.at[pl.ds(off, _B2)], srcb)
        pltpu.sync_copy(dst_hbm.at[pl.ds(off, _B2)], dstb)
        pltpu.async_copy(g0_hbm.at[srcb], rows, sem).wait()
        pltpu.sync_copy(rows, acc.at[dstb], add=True)
    plsc.subcore_barrier()
    pltpu.sync_copy(acc.at[pl.ds(zs, NP // NT), :],
                    out_hbm.at[c, pl.ds(zs, NP // NT), :])


_agg2_kernel = functools.partial(
    pl.kernel,
    out_type=_sds((NC, NP, 16)),
    mesh=plsc.VectorSubcoreMesh(**_MESH),
    compiler_params=pltpu.CompilerParams(use_tc_tiling_on_sc=False, needs_layout_passes=False),
    scratch_types=[
        pltpu.VMEM_SHARED((NP, 16), jnp.float32),
        pltpu.VMEM((_B2,), jnp.int32),
        pltpu.VMEM((_B2,), jnp.int32),
        pltpu.VMEM((_B2, 16), jnp.float32),
        pltpu.SemaphoreType.DMA,
    ],
)(_agg2_body)


_B3 = 800
_G = 256
_CQ = 1072
_ET = E // NT
_NB3 = _ET // _B3
_CPT = CHP // NT


def _agg64_body(src_hbm, dst_hbm, g1_hbm, zeros_hbm, out_hbm,
                acc, srcb, dstb, csrc, cldst, fsrc, fldst, rows, sem):
    c = lax.axis_index("c")
    s = lax.axis_index("s")
    lanes = jnp.arange(L, dtype=jnp.int32)

    def fire():
        def cp(i, _):
            fsrc[pl.ds(i * L, L)] = csrc[pl.ds(i * L, L)]
            fldst[pl.ds(i * L, L)] = cldst[pl.ds(i * L, L)]
            return 0
        lax.fori_loop(0, _G // L, cp, 0)
        pltpu.async_copy(g1_hbm.at[fsrc], rows, sem).wait()
        pltpu.sync_copy(rows, acc.at[fldst], add=True)

    for jx in range(2):
        job = c + 2 * jx
        base = job * CH
        zs = s * _CPT
        pltpu.sync_copy(zeros_hbm.at[pl.ds(zs, _CPT), :], acc.at[pl.ds(zs, _CPT), :])
        plsc.subcore_barrier()

        ebase = s * _ET

        def batch_step(b, cnt):
            off = ebase + b * _B3
            pltpu.sync_copy(src_hbm.at[pl.ds(off, _B3)], srcb)
            pltpu.sync_copy(dst_hbm.at[pl.ds(off, _B3)], dstb)

            def filt(i, cnt):
                dstv = dstb[pl.ds(i * L, L)]
                srcv = srcb[pl.ds(i * L, L)]
                loc = dstv - base
                msk = (loc >= 0) & (loc < CH)
                inc = plsc.cumsum(jnp.where(msk, 1, 0).astype(jnp.int32))
                pos = cnt + inc - 1
                plsc.store_scatter(csrc, [pos], srcv, mask=msk)
                plsc.store_scatter(cldst, [pos], jnp.where(msk, loc, 0), mask=msk)
                return cnt + jnp.sum(jnp.where(msk, 1, 0).astype(jnp.int32))

            cnt = lax.fori_loop(0, _B3 // L, filt, cnt)

            def do_fire(cnt):
                fire()
                def sh(i, _):
                    csrc[pl.ds(i * L, L)] = csrc[pl.ds(_G + i * L, L)]
                    cldst[pl.ds(i * L, L)] = cldst[pl.ds(_G + i * L, L)]
                    return 0
                lax.fori_loop(0, _B3 // L, sh, 0)
                return cnt - _G

            return lax.while_loop(lambda cnt: cnt >= _G, do_fire, cnt)

        cnt = lax.fori_loop(0, _NB3, batch_step, jnp.int32(0))

        def pad(i, cnt):
            p = i * L + lanes
            msk = p >= cnt
            plsc.store_scatter(csrc, [p], p + s * _G, mask=msk)
            plsc.store_scatter(cldst, [p], CH + lanes, mask=msk)
            return cnt
        lax.fori_loop(0, _G // L, pad, cnt)
        fire()

        plsc.subcore_barrier()
        ws = s * (CH // NT)
        pltpu.sync_copy(acc.at[pl.ds(ws, CH // NT), :],
                        out_hbm.at[pl.ds(base + ws, CH // NT), :])
        plsc.subcore_barrier()


_agg64_kernel = functools.partial(
    pl.kernel,
    out_type=_sds((NP, 64)),
    mesh=plsc.VectorSubcoreMesh(**_MESH),
    compiler_params=pltpu.CompilerParams(use_tc_tiling_on_sc=False, needs_layout_passes=False),
    scratch_types=[
        pltpu.VMEM_SHARED((CHP, 64), jnp.float32),
        pltpu.VMEM((_B3,), jnp.int32),
        pltpu.VMEM((_B3,), jnp.int32),
        pltpu.VMEM((_CQ,), jnp.int32),
        pltpu.VMEM((_CQ,), jnp.int32),
        pltpu.VMEM((_G,), jnp.int32),
        pltpu.VMEM((_G,), jnp.int32),
        pltpu.VMEM((_G, 64), jnp.float32),
        pltpu.SemaphoreType.DMA,
    ],
)(_agg64_body)


_BN = 2048


def _tc1_body(deg2_ref, x_ref, dinv_ref, g0_ref):
    deg = deg2_ref[0, :, 0] + deg2_ref[1, :, 0] + 1.0
    dinv = lax.rsqrt(deg)
    dinv_ref[...] = dinv
    g = x_ref[...] * dinv[:, None]
    g0_ref[...] = jnp.concatenate(
        [g, jnp.zeros((g.shape[0], 14), jnp.float32)], axis=1)


def _tc1(deg2, x_pad):
    return pl.pallas_call(
        _tc1_body,
        grid=(NP // _BN,),
        in_specs=[
            pl.BlockSpec((NC, _BN, 16), lambda i: (0, i, 0)),
            pl.BlockSpec((_BN, 2), lambda i: (i, 0)),
        ],
        out_specs=[
            pl.BlockSpec((_BN,), lambda i: (i,)),
            pl.BlockSpec((_BN, 16), lambda i: (i, 0)),
        ],
        out_shape=[_sds((NP,)), _sds((NP, 16))],
    )(deg2, x_pad)


def _tc2_body(ag_ref, g0_ref, dinv_ref, w1_ref, b1_ref, g1_ref):
    dinv = dinv_ref[...]
    z = dinv[:, None] * (ag_ref[0, :, 0:2] + ag_ref[1, :, 0:2] + g0_ref[:, 0:2])
    h1 = z[:, 0:1] * w1_ref[0, :][None, :] + z[:, 1:2] * w1_ref[1, :][None, :]
    h1 = jnp.maximum(h1 + b1_ref[...][None, :], 0.0)
    g1_ref[...] = h1 * dinv[:, None]


def _tc2(agg0, g0, dinv, W1, b1):
    return pl.pallas_call(
        _tc2_body,
        grid=(NP // _BN,),
        in_specs=[
            pl.BlockSpec((NC, _BN, 16), lambda i: (0, i, 0)),
            pl.BlockSpec((_BN, 16), lambda i: (i, 0)),
            pl.BlockSpec((_BN,), lambda i: (i,)),
            pl.BlockSpec((2, 64), lambda i: (0, 0)),
            pl.BlockSpec((64,), lambda i: (0,)),
        ],
        out_specs=pl.BlockSpec((_BN, 64), lambda i: (i, 0)),
        out_shape=_sds((NP, 64)),
    )(agg0, g0, dinv, W1, b1)


def _tc3_body(ag_ref, g1_ref, dinv_ref, w2_ref, b2_ref, wfc_ref, bfc_ref, o_ref):
    dinv = dinv_ref[...]
    z = dinv[:, None] * (ag_ref[...] + g1_ref[...])
    h2 = jnp.dot(z, w2_ref[...], preferred_element_type=jnp.float32)
    h2 = jnp.maximum(h2 + b2_ref[...][None, :], 0.0)
    o = jnp.dot(h2, wfc_ref[...], preferred_element_type=jnp.float32)
    o = o[:, 0] + bfc_ref[0]
    o_ref[...] = 1.0 / (1.0 + jnp.exp(-o))


def _tc3(agg1, g1, dinv, W2, b2, Wfc, bfc):
    return pl.pallas_call(
        _tc3_body,
        grid=(NP // _BN,),
        in_specs=[
            pl.BlockSpec((_BN, 64), lambda i: (i, 0)),
            pl.BlockSpec((_BN, 64), lambda i: (i, 0)),
            pl.BlockSpec((_BN,), lambda i: (i,)),
            pl.BlockSpec((64, 64), lambda i: (0, 0)),
            pl.BlockSpec((64,), lambda i: (0,)),
            pl.BlockSpec((64, 1), lambda i: (0, 0)),
            pl.BlockSpec((1,), lambda i: (0,)),
        ],
        out_specs=pl.BlockSpec((_BN,), lambda i: (i,)),
        out_shape=_sds((NP,)),
    )(agg1, g1, dinv, W2, b2, Wfc, bfc)


def kernel(x, edge_index, W1, b1, W2, b2, Wfc, bfc):
    ei = edge_index.astype(jnp.int32)
    src = ei[0]
    dst = ei[1]
    x_pad = jnp.pad(x, ((0, NP - N), (0, 0)))
    zeros16 = jnp.zeros((NP, 16), jnp.float32)
    ones16 = jnp.ones((_B1, 16), jnp.float32)
    zeros64 = jnp.zeros((CHP, 64), jnp.float32)

    deg2 = _deg_kernel(dst, ones16, zeros16)
    dinv, g0 = _tc1(deg2, x_pad)
    agg0 = _agg2_kernel(src, dst, g0, zeros16)
    g1 = _tc2(agg0, g0, dinv, W1, b1)
    agg1 = _agg64_kernel(src, dst, g1, zeros64)
    o = _tc3(agg1, g1, dinv, W2, b2, Wfc, bfc)
    return o[:N]


# speedup vs baseline: 32.9916x; 1.3585x over previous
import functools

import jax
import jax.numpy as jnp
from jax import lax
from jax.experimental import pallas as pl
from jax.experimental.pallas import tpu as pltpu
from jax.experimental.pallas import tpu_sc as plsc

N = 100000
E = 1600000
NP = 102400
CH = 25600
CHP = CH + 16
NC = 2
NT = 16
L = 16

_MESH = dict(core_axis_name="c", subcore_axis_name="s")


def _sds(shape, dtype=jnp.float32):
    return jax.ShapeDtypeStruct(shape, dtype)


_B1 = 1000
_EW1 = E // (NC * NT)


def _deg_body(dst_hbm, ones_hbm, zeros_hbm, out_hbm, acc, idx, ones_v, sem):
    c = lax.axis_index("c")
    s = lax.axis_index("s")
    wid = c * NT + s
    zs = s * (NP // NT)
    pltpu.sync_copy(zeros_hbm.at[pl.ds(zs, NP // NT), :], acc.at[pl.ds(zs, NP // NT), :])
    pltpu.sync_copy(ones_hbm, ones_v)
    plsc.subcore_barrier()
    base = wid * _EW1
    for b in range(_EW1 // _B1):
        pltpu.sync_copy(dst_hbm.at[pl.ds(base + b * _B1, _B1)], idx)
        pltpu.sync_copy(ones_v, acc.at[idx], add=True)
    plsc.subcore_barrier()
    pltpu.sync_copy(acc.at[pl.ds(zs, NP // NT), :],
                    out_hbm.at[c, pl.ds(zs, NP // NT), :])


_deg_kernel = functools.partial(
    pl.kernel,
    out_type=_sds((NC, NP, 16)),
    mesh=plsc.VectorSubcoreMesh(**_MESH),
    compiler_params=pltpu.CompilerParams(use_tc_tiling_on_sc=False, needs_layout_passes=False),
    scratch_types=[
        pltpu.VMEM_SHARED((NP, 16), jnp.float32),
        pltpu.VMEM((_B1,), jnp.int32),
        pltpu.VMEM((_B1, 16), jnp.float32),
        pltpu.SemaphoreType.DMA,
    ],
)(_deg_body)


_B2 = 1000
_EW2 = E // (NC * NT)


def _agg2_body(src_hbm, dst_hbm, g0_hbm, zeros_hbm, out_hbm, acc, srcb, dstb, rows, sem):
    c = lax.axis_index("c")
    s = lax.axis_index("s")
    wid = c * NT + s
    zs = s * (NP // NT)
    pltpu.sync_copy(zeros_hbm.at[pl.ds(zs, NP // NT), :], acc.at[pl.ds(zs, NP // NT), :])
    plsc.subcore_barrier()
    base = wid * _EW2
    for b in range(_EW2 // _B2):
        off = base + b * _B2
        pltpu.sync_copy(src_hbm.at[pl.ds(off, _B2)], srcb)
        pltpu.sync_copy(dst_hbm.at[pl.ds(off, _B2)], dstb)
        pltpu.async_copy(g0_hbm.at[srcb], rows, sem).wait()
        pltpu.sync_copy(rows, acc.at[dstb], add=True)
    plsc.subcore_barrier()
    pltpu.sync_copy(acc.at[pl.ds(zs, NP // NT), :],
                    out_hbm.at[c, pl.ds(zs, NP // NT), :])


_agg2_kernel = functools.partial(
    pl.kernel,
    out_type=_sds((NC, NP, 16)),
    mesh=plsc.VectorSubcoreMesh(**_MESH),
    compiler_params=pltpu.CompilerParams(use_tc_tiling_on_sc=False, needs_layout_passes=False),
    scratch_types=[
        pltpu.VMEM_SHARED((NP, 16), jnp.float32),
        pltpu.VMEM((_B2,), jnp.int32),
        pltpu.VMEM((_B2,), jnp.int32),
        pltpu.VMEM((_B2, 16), jnp.float32),
        pltpu.SemaphoreType.DMA,
    ],
)(_agg2_body)


_B3 = 400
_G = 128
_CQ = 1024
_ET = E // NT
_NB3 = _ET // _B3
_CPT = CHP // NT


def _agg64_body(src_hbm, dst_hbm, g1_hbm, out_hbm,
                acc, srcA, dstA, srcB, dstB, csrc, cldst,
                fsrc0, fldst0, fsrc1, fldst1, rows0, rows1,
                semA, semB, semG0, semG1):
    c = lax.axis_index("c")
    s = lax.axis_index("s")
    lanes = jnp.arange(L, dtype=jnp.int32)
    ebase = s * _ET

    def stage(b, sb, db, sem):
        bb = jnp.where(b < _NB3, b, b - _NB3)
        off = ebase + bb * _B3
        pltpu.async_copy(src_hbm.at[pl.ds(off, _B3)], sb, sem)
        pltpu.async_copy(dst_hbm.at[pl.ds(off, _B3)], db, sem)

    def wait_stage(sb, db, sem):
        pltpu.make_async_copy(src_hbm.at[pl.ds(0, _B3)], sb, sem).wait()
        pltpu.make_async_copy(dst_hbm.at[pl.ds(0, _B3)], db, sem).wait()

    def issue(fs, fl, rw, sem, head):
        def cp(i, _):
            fs[pl.ds(i * L, L)] = csrc[pl.ds(head + i * L, L)]
            fl[pl.ds(i * L, L)] = cldst[pl.ds(head + i * L, L)]
            return 0
        lax.fori_loop(0, _G // L, cp, 0)
        pltpu.async_copy(g1_hbm.at[fs], rw, sem)

    def drain(fs, fl, rw, sem):
        pltpu.make_async_copy(g1_hbm.at[fs], rw, sem).wait()
        pltpu.sync_copy(rw, acc.at[fl], add=True)

    def fire_step(st):
        cnt, fired = st
        k0 = ((fired // _G) & 1) == 0
        head = fired & (_CQ - 1)
        lax.cond(k0,
                 lambda: issue(fsrc0, fldst0, rows0, semG0, head),
                 lambda: issue(fsrc1, fldst1, rows1, semG1, head))

        def drain_prev():
            lax.cond(k0,
                     lambda: drain(fsrc1, fldst1, rows1, semG1),
                     lambda: drain(fsrc0, fldst0, rows0, semG0))
        lax.cond(fired >= _G, drain_prev, lambda: None)
        return (cnt, fired + _G)

    for jx in range(2):
        job = c + 2 * jx
        base = job * CH

        def zr(i, _):
            for k in range(4):
                rows0[i, pl.ds(k * L, L)] = jnp.zeros((L,), jnp.float32)
            return 0
        lax.fori_loop(0, _G, zr, 0)
        zs = s * _CPT
        for k in range(12):
            pltpu.sync_copy(rows0, acc.at[pl.ds(zs + k * _G, _G), :])
        pltpu.sync_copy(rows0.at[pl.ds(0, _CPT - 12 * _G), :],
                        acc.at[pl.ds(zs + 12 * _G, _CPT - 12 * _G), :])
        plsc.subcore_barrier()

        def filt_batch(sb, db, st):
            cnt, fired = st

            def filt(i, cnt):
                dstv = db[pl.ds(i * L, L)]
                srcv = sb[pl.ds(i * L, L)]
                loc = dstv - base
                msk = (loc >= 0) & (loc < CH)
                inc = plsc.cumsum(jnp.where(msk, 1, 0).astype(jnp.int32))
                pos = (cnt + inc - 1) & (_CQ - 1)
                plsc.store_scatter(csrc, [pos], srcv, mask=msk)
                plsc.store_scatter(cldst, [pos], jnp.where(msk, loc, 0), mask=msk)
                return cnt + jnp.sum(jnp.where(msk, 1, 0).astype(jnp.int32))

            cnt = lax.fori_loop(0, _B3 // L, filt, cnt)
            return lax.while_loop(lambda st2: st2[0] - st2[1] >= _G,
                                  fire_step, (cnt, fired))

        stage(jnp.int32(0), srcA, dstA, semA)
        stage(jnp.int32(1), srcB, dstB, semB)

        def super_step(sb_i, st):
            b0 = sb_i * 2
            wait_stage(srcA, dstA, semA)
            st = filt_batch(srcA, dstA, st)
            stage(b0 + 2, srcA, dstA, semA)
            wait_stage(srcB, dstB, semB)
            st = filt_batch(srcB, dstB, st)
            stage(b0 + 3, srcB, dstB, semB)
            return st

        cnt, fired = lax.fori_loop(0, _NB3 // 2, super_step,
                                   (jnp.int32(0), jnp.int32(0)))
        wait_stage(srcA, dstA, semA)
        wait_stage(srcB, dstB, semB)

        def pad(i, _):
            q = fired + i * L + lanes
            msk = q >= cnt
            wp = q & (_CQ - 1)
            plsc.store_scatter(csrc, [wp], (q & 255) + s * 512, mask=msk)
            plsc.store_scatter(cldst, [wp], CH + lanes, mask=msk)
            return 0
        lax.fori_loop(0, _G // L, pad, 0)
        cnt2, fired2 = fire_step((cnt, fired))
        k0f = (((fired2 - _G) // _G) & 1) == 0
        lax.cond(k0f,
                 lambda: drain(fsrc0, fldst0, rows0, semG0),
                 lambda: drain(fsrc1, fldst1, rows1, semG1))

        plsc.subcore_barrier()
        ws = s * (CH // NT)
        pltpu.sync_copy(acc.at[pl.ds(ws, CH // NT), :],
                        out_hbm.at[pl.ds(base + ws, CH // NT), :])
        plsc.subcore_barrier()


_agg64_kernel = functools.partial(
    pl.kernel,
    out_type=_sds((NP, 64)),
    mesh=plsc.VectorSubcoreMesh(**_MESH),
    compiler_params=pltpu.CompilerParams(use_tc_tiling_on_sc=False, needs_layout_passes=False),
    scratch_types=[
        pltpu.VMEM_SHARED((CHP, 64), jnp.float32),
        pltpu.VMEM((_B3,), jnp.int32),
        pltpu.VMEM((_B3,), jnp.int32),
        pltpu.VMEM((_B3,), jnp.int32),
        pltpu.VMEM((_B3,), jnp.int32),
        pltpu.VMEM((_CQ,), jnp.int32),
        pltpu.VMEM((_CQ,), jnp.int32),
        pltpu.VMEM((_G,), jnp.int32),
        pltpu.VMEM((_G,), jnp.int32),
        pltpu.VMEM((_G,), jnp.int32),
        pltpu.VMEM((_G,), jnp.int32),
        pltpu.VMEM((_G, 64), jnp.float32),
        pltpu.VMEM((_G, 64), jnp.float32),
        pltpu.SemaphoreType.DMA,
        pltpu.SemaphoreType.DMA,
        pltpu.SemaphoreType.DMA,
        pltpu.SemaphoreType.DMA,
    ],
)(_agg64_body)


_BN = 2048


def _tc1_body(deg2_ref, x_ref, dinv_ref, g0_ref):
    deg = deg2_ref[0, :, 0] + deg2_ref[1, :, 0] + 1.0
    dinv = lax.rsqrt(deg)
    dinv_ref[...] = dinv
    g = x_ref[...] * dinv[:, None]
    g0_ref[...] = jnp.concatenate(
        [g, jnp.zeros((g.shape[0], 14), jnp.float32)], axis=1)


def _tc1(deg2, x_pad):
    return pl.pallas_call(
        _tc1_body,
        grid=(NP // _BN,),
        in_specs=[
            pl.BlockSpec((NC, _BN, 16), lambda i: (0, i, 0)),
            pl.BlockSpec((_BN, 2), lambda i: (i, 0)),
        ],
        out_specs=[
            pl.BlockSpec((_BN,), lambda i: (i,)),
            pl.BlockSpec((_BN, 16), lambda i: (i, 0)),
        ],
        out_shape=[_sds((NP,)), _sds((NP, 16))],
    )(deg2, x_pad)


def _tc2_body(ag_ref, g0_ref, dinv_ref, w1_ref, b1_ref, g1_ref):
    dinv = dinv_ref[...]
    z = dinv[:, None] * (ag_ref[0, :, 0:2] + ag_ref[1, :, 0:2] + g0_ref[:, 0:2])
    h1 = z[:, 0:1] * w1_ref[0, :][None, :] + z[:, 1:2] * w1_ref[1, :][None, :]
    h1 = jnp.maximum(h1 + b1_ref[...][None, :], 0.0)
    g1_ref[...] = h1 * dinv[:, None]


def _tc2(agg0, g0, dinv, W1, b1):
    return pl.pallas_call(
        _tc2_body,
        grid=(NP // _BN,),
        in_specs=[
            pl.BlockSpec((NC, _BN, 16), lambda i: (0, i, 0)),
            pl.BlockSpec((_BN, 16), lambda i: (i, 0)),
            pl.BlockSpec((_BN,), lambda i: (i,)),
            pl.BlockSpec((2, 64), lambda i: (0, 0)),
            pl.BlockSpec((64,), lambda i: (0,)),
        ],
        out_specs=pl.BlockSpec((_BN, 64), lambda i: (i, 0)),
        out_shape=_sds((NP, 64)),
    )(agg0, g0, dinv, W1, b1)


def _tc3_body(ag_ref, g1_ref, dinv_ref, w2_ref, b2_ref, wfc_ref, bfc_ref, o_ref):
    dinv = dinv_ref[...]
    z = dinv[:, None] * (ag_ref[...] + g1_ref[...])
    h2 = jnp.dot(z, w2_ref[...], preferred_element_type=jnp.float32)
    h2 = jnp.maximum(h2 + b2_ref[...][None, :], 0.0)
    o = jnp.dot(h2, wfc_ref[...], preferred_element_type=jnp.float32)
    o = o[:, 0] + bfc_ref[0]
    o_ref[...] = 1.0 / (1.0 + jnp.exp(-o))


def _tc3(agg1, g1, dinv, W2, b2, Wfc, bfc):
    return pl.pallas_call(
        _tc3_body,
        grid=(NP // _BN,),
        in_specs=[
            pl.BlockSpec((_BN, 64), lambda i: (i, 0)),
            pl.BlockSpec((_BN, 64), lambda i: (i, 0)),
            pl.BlockSpec((_BN,), lambda i: (i,)),
            pl.BlockSpec((64, 64), lambda i: (0, 0)),
            pl.BlockSpec((64,), lambda i: (0,)),
            pl.BlockSpec((64, 1), lambda i: (0, 0)),
            pl.BlockSpec((1,), lambda i: (0,)),
        ],
        out_specs=pl.BlockSpec((_BN,), lambda i: (i,)),
        out_shape=_sds((NP,)),
    )(agg1, g1, dinv, W2, b2, Wfc, bfc)


def kernel(x, edge_index, W1, b1, W2, b2, Wfc, bfc):
    ei = edge_index.astype(jnp.int32)
    src = ei[0]
    dst = ei[1]
    x_pad = jnp.pad(x, ((0, NP - N), (0, 0)))
    zeros16 = jnp.zeros((NP, 16), jnp.float32)
    ones16 = jnp.ones((_B1, 16), jnp.float32)

    deg2 = _deg_kernel(dst, ones16, zeros16)
    dinv, g0 = _tc1(deg2, x_pad)
    agg0 = _agg2_kernel(src, dst, g0, zeros16)
    g1 = _tc2(agg0, g0, dinv, W1, b1)
    agg1 = _agg64_kernel(src, dst, g1)
    o = _tc3(agg1, g1, dinv, W2, b2, Wfc, bfc)
    return o[:N]


# speedup vs baseline: 35.3474x; 1.0714x over previous
import functools

import jax
import jax.numpy as jnp
from jax import lax
from jax.experimental import pallas as pl
from jax.experimental.pallas import tpu as pltpu
from jax.experimental.pallas import tpu_sc as plsc

N = 100000
E = 1600000
NP = 102400
CH = 25600
CHP = CH + 16
NC = 2
NT = 16
L = 16

_MESH = dict(core_axis_name="c", subcore_axis_name="s")


def _sds(shape, dtype=jnp.float32):
    return jax.ShapeDtypeStruct(shape, dtype)


_B1 = 1000
_EW1 = E // (NC * NT)
_NB1 = _EW1 // _B1


def _deg_body(dst_hbm, out_hbm, acc, idxA, idxB, ones_v, zbuf, semA, semB,
              semSA, semSB):
    c = lax.axis_index("c")
    s = lax.axis_index("s")
    wid = c * NT + s
    ebase = wid * _EW1

    def fill(i, _):
        ones_v[i, :] = jnp.ones((L,), jnp.float32)
        return 0
    lax.fori_loop(0, _B1, fill, 0)

    def zfill(i, _):
        zbuf[i, :] = jnp.zeros((L,), jnp.float32)
        return 0
    lax.fori_loop(0, 400, zfill, 0)
    zs = s * (NP // NT)
    for k in range(16):
        pltpu.sync_copy(zbuf, acc.at[pl.ds(zs + k * 400, 400), :])
    plsc.subcore_barrier()

    def stage(b, idx, sem):
        bb = jnp.where(b < _NB1, b, b - _NB1)
        pltpu.async_copy(dst_hbm.at[pl.ds(ebase + bb * _B1, _B1)], idx, sem)

    def wait_stage(idx, sem):
        pltpu.make_async_copy(dst_hbm.at[pl.ds(0, _B1)], idx, sem).wait()

    def drain_scat(idx, semS):
        pltpu.make_async_copy(ones_v, acc.at[idx], semS).wait()

    def half(b, idx_t, sem_t, semS_t, idx_o, sem_o, semS_o):
        wait_stage(idx_t, sem_t)
        pltpu.async_copy(ones_v, acc.at[idx_t], semS_t, add=True)

        def dr():
            drain_scat(idx_o, semS_o)
        lax.cond(b >= 1, dr, lambda: None)
        stage(b + 1, idx_o, sem_o)

    stage(jnp.int32(0), idxA, semA)

    def step(b, _):
        lax.cond((b & 1) == 0,
                 lambda: half(b, idxA, semA, semSA, idxB, semB, semSB),
                 lambda: half(b, idxB, semB, semSB, idxA, semA, semSA))
        return 0
    lax.fori_loop(0, _NB1, step, 0)
    lax.cond(((_NB1 - 1) & 1) == 0,
             lambda: drain_scat(idxA, semSA),
             lambda: drain_scat(idxB, semSB))
    wait_stage(idxA if _NB1 % 2 == 0 else idxB,
               semA if _NB1 % 2 == 0 else semB)

    plsc.subcore_barrier()
    pltpu.sync_copy(acc.at[pl.ds(zs, NP // NT), :],
                    out_hbm.at[c, pl.ds(zs, NP // NT), :])


_deg_kernel = functools.partial(
    pl.kernel,
    out_type=_sds((NC, NP, 16)),
    mesh=plsc.VectorSubcoreMesh(**_MESH),
    compiler_params=pltpu.CompilerParams(use_tc_tiling_on_sc=False, needs_layout_passes=False),
    scratch_types=[
        pltpu.VMEM_SHARED((NP, 16), jnp.float32),
        pltpu.VMEM((_B1,), jnp.int32),
        pltpu.VMEM((_B1,), jnp.int32),
        pltpu.VMEM((_B1, 16), jnp.float32),
        pltpu.VMEM((400, 16), jnp.float32),
        pltpu.SemaphoreType.DMA,
        pltpu.SemaphoreType.DMA,
        pltpu.SemaphoreType.DMA,
        pltpu.SemaphoreType.DMA,
    ],
)(_deg_body)


_B2 = 400
_EW2 = E // (NC * NT)
_NB2 = _EW2 // _B2


def _agg2_body(src_hbm, dst_hbm, g0_hbm, out_hbm, acc,
               srcA, dstA, srcB, dstB, rows0, rows1,
               semA, semB, semG0, semG1):
    c = lax.axis_index("c")
    s = lax.axis_index("s")
    wid = c * NT + s
    ebase = wid * _EW2

    def zr(i, _):
        rows0[i, :] = jnp.zeros((L,), jnp.float32)
        return 0
    lax.fori_loop(0, _B2, zr, 0)
    zs = s * (NP // NT)
    for k in range(NP // NT // _B2):
        pltpu.sync_copy(rows0, acc.at[pl.ds(zs + k * _B2, _B2), :])
    plsc.subcore_barrier()

    def stage(b, sb, db, sem):
        bb = jnp.where(b < _NB2, b, b - _NB2)
        off = ebase + bb * _B2
        pltpu.async_copy(src_hbm.at[pl.ds(off, _B2)], sb, sem)
        pltpu.async_copy(dst_hbm.at[pl.ds(off, _B2)], db, sem)

    def wait_stage(sb, db, sem):
        pltpu.make_async_copy(src_hbm.at[pl.ds(0, _B2)], sb, sem).wait()
        pltpu.make_async_copy(dst_hbm.at[pl.ds(0, _B2)], db, sem).wait()

    def drain(sb, db, rw, semG):
        pltpu.make_async_copy(g0_hbm.at[sb], rw, semG).wait()
        pltpu.sync_copy(rw, acc.at[db], add=True)

    def half(b, sb_t, db_t, rw_t, sem_t, semG_t, sb_o, db_o, rw_o, sem_o, semG_o):
        wait_stage(sb_t, db_t, sem_t)
        pltpu.async_copy(g0_hbm.at[sb_t], rw_t, semG_t)

        def dr():
            drain(sb_o, db_o, rw_o, semG_o)
        lax.cond(b >= 1, dr, lambda: None)
        stage(b + 1, sb_o, db_o, sem_o)

    stage(jnp.int32(0), srcA, dstA, semA)

    def step(b, _):
        lax.cond((b & 1) == 0,
                 lambda: half(b, srcA, dstA, rows0, semA, semG0,
                              srcB, dstB, rows1, semB, semG1),
                 lambda: half(b, srcB, dstB, rows1, semB, semG1,
                              srcA, dstA, rows0, semA, semG0))
        return 0
    lax.fori_loop(0, _NB2, step, 0)
    lax.cond(((_NB2 - 1) & 1) == 0,
             lambda: drain(srcA, dstA, rows0, semG0),
             lambda: drain(srcB, dstB, rows1, semG1))
    wait_stage(srcA if _NB2 % 2 == 0 else srcB,
               dstA if _NB2 % 2 == 0 else dstB,
               semA if _NB2 % 2 == 0 else semB)

    plsc.subcore_barrier()
    pltpu.sync_copy(acc.at[pl.ds(zs, NP // NT), :],
                    out_hbm.at[c, pl.ds(zs, NP // NT), :])


_agg2_kernel = functools.partial(
    pl.kernel,
    out_type=_sds((NC, NP, 16)),
    mesh=plsc.VectorSubcoreMesh(**_MESH),
    compiler_params=pltpu.CompilerParams(use_tc_tiling_on_sc=False, needs_layout_passes=False),
    scratch_types=[
        pltpu.VMEM_SHARED((NP, 16), jnp.float32),
        pltpu.VMEM((_B2,), jnp.int32),
        pltpu.VMEM((_B2,), jnp.int32),
        pltpu.VMEM((_B2,), jnp.int32),
        pltpu.VMEM((_B2,), jnp.int32),
        pltpu.VMEM((_B2, 16), jnp.float32),
        pltpu.VMEM((_B2, 16), jnp.float32),
        pltpu.SemaphoreType.DMA,
        pltpu.SemaphoreType.DMA,
        pltpu.SemaphoreType.DMA,
        pltpu.SemaphoreType.DMA,
    ],
)(_agg2_body)


_B3 = 400
_G = 128
_CQ = 1024
_ET = E // NT
_NB3 = _ET // _B3
_CPT = CHP // NT


def _agg64_body(src_hbm, dst_hbm, g1_hbm, out_hbm,
                acc, srcA, dstA, srcB, dstB, csrc, cldst,
                fsrc0, fldst0, fsrc1, fldst1, rows0, rows1,
                semA, semB, semG0, semG1):
    c = lax.axis_index("c")
    s = lax.axis_index("s")
    lanes = jnp.arange(L, dtype=jnp.int32)
    ebase = s * _ET

    def stage(b, sb, db, sem):
        bb = jnp.where(b < _NB3, b, b - _NB3)
        off = ebase + bb * _B3
        pltpu.async_copy(src_hbm.at[pl.ds(off, _B3)], sb, sem)
        pltpu.async_copy(dst_hbm.at[pl.ds(off, _B3)], db, sem)

    def wait_stage(sb, db, sem):
        pltpu.make_async_copy(src_hbm.at[pl.ds(0, _B3)], sb, sem).wait()
        pltpu.make_async_copy(dst_hbm.at[pl.ds(0, _B3)], db, sem).wait()

    def issue(fs, fl, rw, sem, head):
        def cp(i, _):
            fs[pl.ds(i * L, L)] = csrc[pl.ds(head + i * L, L)]
            fl[pl.ds(i * L, L)] = cldst[pl.ds(head + i * L, L)]
            return 0
        lax.fori_loop(0, _G // L, cp, 0)
        pltpu.async_copy(g1_hbm.at[fs], rw, sem)

    def drain(fs, fl, rw, sem):
        pltpu.make_async_copy(g1_hbm.at[fs], rw, sem).wait()
        pltpu.sync_copy(rw, acc.at[fl], add=True)

    def fire_step(st):
        cnt, fired = st
        k0 = ((fired // _G) & 1) == 0
        head = fired & (_CQ - 1)
        lax.cond(k0,
                 lambda: issue(fsrc0, fldst0, rows0, semG0, head),
                 lambda: issue(fsrc1, fldst1, rows1, semG1, head))

        def drain_prev():
            lax.cond(k0,
                     lambda: drain(fsrc1, fldst1, rows1, semG1),
                     lambda: drain(fsrc0, fldst0, rows0, semG0))
        lax.cond(fired >= _G, drain_prev, lambda: None)
        return (cnt, fired + _G)

    for jx in range(2):
        job = c + 2 * jx
        base = job * CH

        def zr(i, _):
            for k in range(4):
                rows0[i, pl.ds(k * L, L)] = jnp.zeros((L,), jnp.float32)
            return 0
        lax.fori_loop(0, _G, zr, 0)
        zs = s * _CPT
        for k in range(12):
            pltpu.sync_copy(rows0, acc.at[pl.ds(zs + k * _G, _G), :])
        pltpu.sync_copy(rows0.at[pl.ds(0, _CPT - 12 * _G), :],
                        acc.at[pl.ds(zs + 12 * _G, _CPT - 12 * _G), :])
        plsc.subcore_barrier()

        def filt_batch(sb, db, st):
            cnt, fired = st

            def filt(i, cnt):
                dstv = db[pl.ds(i * L, L)]
                srcv = sb[pl.ds(i * L, L)]
                loc = dstv - base
                msk = (loc >= 0) & (loc < CH)
                inc = plsc.cumsum(jnp.where(msk, 1, 0).astype(jnp.int32))
                pos = (cnt + inc - 1) & (_CQ - 1)
                plsc.store_scatter(csrc, [pos], srcv, mask=msk)
                plsc.store_scatter(cldst, [pos], jnp.where(msk, loc, 0), mask=msk)
                return cnt + jnp.sum(jnp.where(msk, 1, 0).astype(jnp.int32))

            cnt = lax.fori_loop(0, _B3 // L, filt, cnt)
            return lax.while_loop(lambda st2: st2[0] - st2[1] >= _G,
                                  fire_step, (cnt, fired))

        stage(jnp.int32(0), srcA, dstA, semA)
        stage(jnp.int32(1), srcB, dstB, semB)

        def super_step(sb_i, st):
            b0 = sb_i * 2
            wait_stage(srcA, dstA, semA)
            st = filt_batch(srcA, dstA, st)
            stage(b0 + 2, srcA, dstA, semA)
            wait_stage(srcB, dstB, semB)
            st = filt_batch(srcB, dstB, st)
            stage(b0 + 3, srcB, dstB, semB)
            return st

        cnt, fired = lax.fori_loop(0, _NB3 // 2, super_step,
                                   (jnp.int32(0), jnp.int32(0)))
        wait_stage(srcA, dstA, semA)
        wait_stage(srcB, dstB, semB)

        def pad(i, _):
            q = fired + i * L + lanes
            msk = q >= cnt
            wp = q & (_CQ - 1)
            plsc.store_scatter(csrc, [wp], (q & 255) + s * 512, mask=msk)
            plsc.store_scatter(cldst, [wp], CH + lanes, mask=msk)
            return 0
        lax.fori_loop(0, _G // L, pad, 0)
        cnt2, fired2 = fire_step((cnt, fired))
        k0f = (((fired2 - _G) // _G) & 1) == 0
        lax.cond(k0f,
                 lambda: drain(fsrc0, fldst0, rows0, semG0),
                 lambda: drain(fsrc1, fldst1, rows1, semG1))

        plsc.subcore_barrier()
        ws = s * (CH // NT)
        pltpu.sync_copy(acc.at[pl.ds(ws, CH // NT), :],
                        out_hbm.at[pl.ds(base + ws, CH // NT), :])
        plsc.subcore_barrier()


_agg64_kernel = functools.partial(
    pl.kernel,
    out_type=_sds((NP, 64)),
    mesh=plsc.VectorSubcoreMesh(**_MESH),
    compiler_params=pltpu.CompilerParams(use_tc_tiling_on_sc=False, needs_layout_passes=False),
    scratch_types=[
        pltpu.VMEM_SHARED((CHP, 64), jnp.float32),
        pltpu.VMEM((_B3,), jnp.int32),
        pltpu.VMEM((_B3,), jnp.int32),
        pltpu.VMEM((_B3,), jnp.int32),
        pltpu.VMEM((_B3,), jnp.int32),
        pltpu.VMEM((_CQ,), jnp.int32),
        pltpu.VMEM((_CQ,), jnp.int32),
        pltpu.VMEM((_G,), jnp.int32),
        pltpu.VMEM((_G,), jnp.int32),
        pltpu.VMEM((_G,), jnp.int32),
        pltpu.VMEM((_G,), jnp.int32),
        pltpu.VMEM((_G, 64), jnp.float32),
        pltpu.VMEM((_G, 64), jnp.float32),
        pltpu.SemaphoreType.DMA,
        pltpu.SemaphoreType.DMA,
        pltpu.SemaphoreType.DMA,
        pltpu.SemaphoreType.DMA,
    ],
)(_agg64_body)


_BN = 2048


def _tc1_body(deg2_ref, x_ref, dinv_ref, g0_ref):
    deg = deg2_ref[0, :, 0] + deg2_ref[1, :, 0] + 1.0
    dinv = lax.rsqrt(deg)
    dinv_ref[...] = dinv
    g = x_ref[...] * dinv[:, None]
    g0_ref[...] = jnp.concatenate(
        [g, jnp.zeros((g.shape[0], 14), jnp.float32)], axis=1)


def _tc1(deg2, x_pad):
    return pl.pallas_call(
        _tc1_body,
        grid=(NP // _BN,),
        in_specs=[
            pl.BlockSpec((NC, _BN, 16), lambda i: (0, i, 0)),
            pl.BlockSpec((_BN, 2), lambda i: (i, 0)),
        ],
        out_specs=[
            pl.BlockSpec((_BN,), lambda i: (i,)),
            pl.BlockSpec((_BN, 16), lambda i: (i, 0)),
        ],
        out_shape=[_sds((NP,)), _sds((NP, 16))],
    )(deg2, x_pad)


def _tc2_body(ag_ref, g0_ref, dinv_ref, w1_ref, b1_ref, g1_ref):
    dinv = dinv_ref[...]
    z = dinv[:, None] * (ag_ref[0, :, 0:2] + ag_ref[1, :, 0:2] + g0_ref[:, 0:2])
    h1 = z[:, 0:1] * w1_ref[0, :][None, :] + z[:, 1:2] * w1_ref[1, :][None, :]
    h1 = jnp.maximum(h1 + b1_ref[...][None, :], 0.0)
    g1_ref[...] = h1 * dinv[:, None]


def _tc2(agg0, g0, dinv, W1, b1):
    return pl.pallas_call(
        _tc2_body,
        grid=(NP // _BN,),
        in_specs=[
            pl.BlockSpec((NC, _BN, 16), lambda i: (0, i, 0)),
            pl.BlockSpec((_BN, 16), lambda i: (i, 0)),
            pl.BlockSpec((_BN,), lambda i: (i,)),
            pl.BlockSpec((2, 64), lambda i: (0, 0)),
            pl.BlockSpec((64,), lambda i: (0,)),
        ],
        out_specs=pl.BlockSpec((_BN, 64), lambda i: (i, 0)),
        out_shape=_sds((NP, 64)),
    )(agg0, g0, dinv, W1, b1)


def _tc3_body(ag_ref, g1_ref, dinv_ref, w2_ref, b2_ref, wfc_ref, bfc_ref, o_ref):
    dinv = dinv_ref[...]
    z = dinv[:, None] * (ag_ref[...] + g1_ref[...])
    h2 = jnp.dot(z, w2_ref[...], preferred_element_type=jnp.float32)
    h2 = jnp.maximum(h2 + b2_ref[...][None, :], 0.0)
    o = jnp.dot(h2, wfc_ref[...], preferred_element_type=jnp.float32)
    o = o[:, 0] + bfc_ref[0]
    o_ref[...] = 1.0 / (1.0 + jnp.exp(-o))


def _tc3(agg1, g1, dinv, W2, b2, Wfc, bfc):
    return pl.pallas_call(
        _tc3_body,
        grid=(NP // _BN,),
        in_specs=[
            pl.BlockSpec((_BN, 64), lambda i: (i, 0)),
            pl.BlockSpec((_BN, 64), lambda i: (i, 0)),
            pl.BlockSpec((_BN,), lambda i: (i,)),
            pl.BlockSpec((64, 64), lambda i: (0, 0)),
            pl.BlockSpec((64,), lambda i: (0,)),
            pl.BlockSpec((64, 1), lambda i: (0, 0)),
            pl.BlockSpec((1,), lambda i: (0,)),
        ],
        out_specs=pl.BlockSpec((_BN,), lambda i: (i,)),
        out_shape=_sds((NP,)),
    )(agg1, g1, dinv, W2, b2, Wfc, bfc)


def kernel(x, edge_index, W1, b1, W2, b2, Wfc, bfc):
    ei = edge_index.astype(jnp.int32)
    src = ei[0]
    dst = ei[1]
    x_pad = jnp.pad(x, ((0, NP - N), (0, 0)))
    deg2 = _deg_kernel(dst)
    dinv, g0 = _tc1(deg2, x_pad)
    agg0 = _agg2_kernel(src, dst, g0)
    g1 = _tc2(agg0, g0, dinv, W1, b1)
    agg1 = _agg64_kernel(src, dst, g1)
    o = _tc3(agg1, g1, dinv, W2, b2, Wfc, bfc)
    return o[:N]


# speedup vs baseline: 43.1692x; 1.2213x over previous
import functools

import jax
import jax.numpy as jnp
from jax import lax
from jax.experimental import pallas as pl
from jax.experimental.pallas import tpu as pltpu
from jax.experimental.pallas import tpu_sc as plsc

N = 100000
E = 1600000
NP = 102400
CH = 25600
CHP = CH + 16
NC = 2
NT = 16
L = 16

_MESH = dict(core_axis_name="c", subcore_axis_name="s",
             num_cores=NC, num_subcores=NT)


def _sds(shape, dtype=jnp.float32):
    return jax.ShapeDtypeStruct(shape, dtype)


_B1 = 1000
_ED = E // NT
_NBD = _ED // _B1
_NSL = NP // (NC * NT)
_DC = 160


def _deg_body(dst_hbm, x0_hbm, x1_hbm, g0_hbm, dinv_hbm, acc,
              idxA, idxB, ones_v, dbuf, x0b, x1b, g0b, dvb,
              semA, semB, semSA, semSB):
    c = lax.axis_index("c")
    s = lax.axis_index("s")
    lanes = jnp.arange(L, dtype=jnp.int32)
    ebase = s * _ED

    def fill(i, _):
        ones_v[i, :] = jnp.ones((L,), jnp.float32)
        return 0
    lax.fori_loop(0, _B1, fill, 0)

    def zfill(i, _):
        g0b[i, :] = jnp.zeros((L,), jnp.float32)
        return 0
    lax.fori_loop(0, _DC, zfill, 0)
    zs = s * (NP // NT)
    for k in range(NP // NT // _DC):
        pltpu.sync_copy(g0b, acc.at[pl.ds(zs + k * _DC, _DC), :])
    plsc.subcore_barrier()

    def stage(b, idx, sem):
        bb = jnp.where(b < _NBD, b, b - _NBD)
        pltpu.async_copy(dst_hbm.at[pl.ds(ebase + bb * _B1, _B1)], idx, sem)

    def wait_stage(idx, sem):
        pltpu.make_async_copy(dst_hbm.at[pl.ds(0, _B1)], idx, sem).wait()

    def drain_scat(idx, semS):
        pltpu.make_async_copy(ones_v, acc.at[idx], semS).wait()

    def half(b, idx_t, sem_t, semS_t, idx_o, sem_o, semS_o):
        wait_stage(idx_t, sem_t)
        pltpu.async_copy(ones_v, acc.at[idx_t], semS_t, add=True)

        def dr():
            drain_scat(idx_o, semS_o)
        lax.cond(b >= 1, dr, lambda: None)
        stage(b + 1, idx_o, sem_o)

    stage(jnp.int32(0), idxA, semA)

    def step(b, _):
        lax.cond((b & 1) == 0,
                 lambda: half(b, idxA, semA, semSA, idxB, semB, semSB),
                 lambda: half(b, idxB, semB, semSB, idxA, semA, semSA))
        return 0
    lax.fori_loop(0, _NBD, step, 0)
    lax.cond(((_NBD - 1) & 1) == 0,
             lambda: drain_scat(idxA, semSA),
             lambda: drain_scat(idxB, semSB))
    wait_stage(idxA if _NBD % 2 == 0 else idxB,
               semA if _NBD % 2 == 0 else semB)

    plsc.subcore_barrier()

    nbase = (c * NT + s) * _NSL
    col0 = jnp.zeros((L,), jnp.int32)
    col1 = jnp.full((L,), 1, jnp.int32)
    for ch in range(_NSL // _DC):
        nb = nbase + ch * _DC
        pltpu.sync_copy(acc.at[pl.ds(nb, _DC), :], dbuf)
        pltpu.sync_copy(x0_hbm.at[pl.ds(nb, _DC)], x0b)
        pltpu.sync_copy(x1_hbm.at[pl.ds(nb, _DC)], x1b)

        def grp(g, _):
            ridx = g * L + lanes
            dv = plsc.load_gather(dbuf, [ridx, col0]) + 1.0
            iv = plsc.bitcast(dv, jnp.int32)
            y = plsc.bitcast(jnp.int32(0x5F3759DF) - (iv >> 1), jnp.float32)
            hd = 0.5 * dv
            y = y * (1.5 - hd * y * y)
            y = y * (1.5 - hd * y * y)
            y = y * (1.5 - hd * y * y)
            x0v = x0b[pl.ds(g * L, L)]
            x1v = x1b[pl.ds(g * L, L)]
            plsc.store_scatter(g0b, [ridx, col0], x0v * y)
            plsc.store_scatter(g0b, [ridx, col1], x1v * y)
            dvb[pl.ds(g * L, L)] = y
            return 0
        lax.fori_loop(0, _DC // L, grp, 0)
        pltpu.sync_copy(g0b, g0_hbm.at[pl.ds(nb, _DC), :])
        pltpu.sync_copy(dvb, dinv_hbm.at[pl.ds(nb, _DC)])


_deg_kernel = functools.partial(
    pl.kernel,
    out_type=(_sds((NP, 16)), _sds((NP,))),
    mesh=plsc.VectorSubcoreMesh(**_MESH),
    compiler_params=pltpu.CompilerParams(use_tc_tiling_on_sc=False, needs_layout_passes=False),
    scratch_types=[
        pltpu.VMEM_SHARED((NP, 16), jnp.float32),
        pltpu.VMEM((_B1,), jnp.int32),
        pltpu.VMEM((_B1,), jnp.int32),
        pltpu.VMEM((_B1, 16), jnp.float32),
        pltpu.VMEM((_DC, 16), jnp.float32),
        pltpu.VMEM((_DC,), jnp.float32),
        pltpu.VMEM((_DC,), jnp.float32),
        pltpu.VMEM((_DC, 16), jnp.float32),
        pltpu.VMEM((_DC,), jnp.float32),
        pltpu.SemaphoreType.DMA,
        pltpu.SemaphoreType.DMA,
        pltpu.SemaphoreType.DMA,
        pltpu.SemaphoreType.DMA,
    ],
)(_deg_body)


_B2 = 400
_EW2 = E // (NC * NT)
_NB2 = _EW2 // _B2


def _agg2_body(src_hbm, dst_hbm, g0_hbm, out_hbm, acc,
               srcA, dstA, srcB, dstB, rows0, rows1,
               semA, semB, semG0, semG1):
    c = lax.axis_index("c")
    s = lax.axis_index("s")
    wid = c * NT + s
    ebase = wid * _EW2

    def zr(i, _):
        rows0[i, :] = jnp.zeros((L,), jnp.float32)
        return 0
    lax.fori_loop(0, _B2, zr, 0)
    zs = s * (NP // NT)

    def init_g0():
        for k in range(NP // NT // _B2):
            pltpu.sync_copy(g0_hbm.at[pl.ds(zs + k * _B2, _B2), :],
                            acc.at[pl.ds(zs + k * _B2, _B2), :])

    def init_zero():
        for k in range(NP // NT // _B2):
            pltpu.sync_copy(rows0, acc.at[pl.ds(zs + k * _B2, _B2), :])
    lax.cond(c == 0, init_g0, init_zero)
    plsc.subcore_barrier()

    def stage(b, sb, db, sem):
        bb = jnp.where(b < _NB2, b, b - _NB2)
        off = ebase + bb * _B2
        pltpu.async_copy(src_hbm.at[pl.ds(off, _B2)], sb, sem)
        pltpu.async_copy(dst_hbm.at[pl.ds(off, _B2)], db, sem)

    def wait_stage(sb, db, sem):
        pltpu.make_async_copy(src_hbm.at[pl.ds(0, _B2)], sb, sem).wait()
        pltpu.make_async_copy(dst_hbm.at[pl.ds(0, _B2)], db, sem).wait()

    def drain(sb, db, rw, semG):
        pltpu.make_async_copy(g0_hbm.at[sb], rw, semG).wait()
        pltpu.sync_copy(rw, acc.at[db], add=True)

    def half(b, sb_t, db_t, rw_t, sem_t, semG_t, sb_o, db_o, rw_o, sem_o, semG_o):
        wait_stage(sb_t, db_t, sem_t)
        pltpu.async_copy(g0_hbm.at[sb_t], rw_t, semG_t)

        def dr():
            drain(sb_o, db_o, rw_o, semG_o)
        lax.cond(b >= 1, dr, lambda: None)
        stage(b + 1, sb_o, db_o, sem_o)

    stage(jnp.int32(0), srcA, dstA, semA)

    def step(b, _):
        lax.cond((b & 1) == 0,
                 lambda: half(b, srcA, dstA, rows0, semA, semG0,
                              srcB, dstB, rows1, semB, semG1),
                 lambda: half(b, srcB, dstB, rows1, semB, semG1,
                              srcA, dstA, rows0, semA, semG0))
        return 0
    lax.fori_loop(0, _NB2, step, 0)
    lax.cond(((_NB2 - 1) & 1) == 0,
             lambda: drain(srcA, dstA, rows0, semG0),
             lambda: drain(srcB, dstB, rows1, semG1))
    wait_stage(srcA if _NB2 % 2 == 0 else srcB,
               dstA if _NB2 % 2 == 0 else dstB,
               semA if _NB2 % 2 == 0 else semB)

    plsc.subcore_barrier()
    pltpu.sync_copy(acc.at[pl.ds(zs, NP // NT), :],
                    out_hbm.at[c, pl.ds(zs, NP // NT), :])


_agg2_kernel = functools.partial(
    pl.kernel,
    out_type=_sds((NC, NP, 16)),
    mesh=plsc.VectorSubcoreMesh(**_MESH),
    compiler_params=pltpu.CompilerParams(use_tc_tiling_on_sc=False, needs_layout_passes=False),
    scratch_types=[
        pltpu.VMEM_SHARED((NP, 16), jnp.float32),
        pltpu.VMEM((_B2,), jnp.int32),
        pltpu.VMEM((_B2,), jnp.int32),
        pltpu.VMEM((_B2,), jnp.int32),
        pltpu.VMEM((_B2,), jnp.int32),
        pltpu.VMEM((_B2, 16), jnp.float32),
        pltpu.VMEM((_B2, 16), jnp.float32),
        pltpu.SemaphoreType.DMA,
        pltpu.SemaphoreType.DMA,
        pltpu.SemaphoreType.DMA,
        pltpu.SemaphoreType.DMA,
    ],
)(_agg2_body)


_B3 = 400
_G = 128
_CQ = 1024
_ET = E // NT
_NB3 = _ET // _B3
_CPT = CHP // NT


def _agg64_body(src_hbm, dst_hbm, g1_hbm, out_hbm,
                acc, srcA, dstA, srcB, dstB, csrc, cldst,
                fsrc0, fldst0, fsrc1, fldst1, rows0, rows1,
                semA, semB, semG0, semG1, semS0, semS1):
    c = lax.axis_index("c")
    s = lax.axis_index("s")
    lanes = jnp.arange(L, dtype=jnp.int32)
    ebase = s * _ET

    def stage(b, sb, db, sem):
        bb = jnp.where(b < _NB3, b, b - _NB3)
        off = ebase + bb * _B3
        pltpu.async_copy(src_hbm.at[pl.ds(off, _B3)], sb, sem)
        pltpu.async_copy(dst_hbm.at[pl.ds(off, _B3)], db, sem)

    def wait_stage(sb, db, sem):
        pltpu.make_async_copy(src_hbm.at[pl.ds(0, _B3)], sb, sem).wait()
        pltpu.make_async_copy(dst_hbm.at[pl.ds(0, _B3)], db, sem).wait()

    def issue(fs, fl, rw, sem, head):
        def cp(i, _):
            fs[pl.ds(i * L, L)] = csrc[pl.ds(head + i * L, L)]
            fl[pl.ds(i * L, L)] = cldst[pl.ds(head + i * L, L)]
            return 0
        lax.fori_loop(0, _G // L, cp, 0)
        pltpu.async_copy(g1_hbm.at[fs], rw, sem)

    def drain(fs, fl, rw, sem, semS):
        pltpu.make_async_copy(g1_hbm.at[fs], rw, sem).wait()
        pltpu.async_copy(rw, acc.at[fl], semS, add=True)

    def wait_scat(fl, rw, semS):
        pltpu.make_async_copy(rw, acc.at[fl], semS).wait()

    def fire_step(st):
        cnt, fired = st
        k0 = ((fired // _G) & 1) == 0
        head = fired & (_CQ - 1)

        def wait_same():
            lax.cond(k0,
                     lambda: wait_scat(fldst0, rows0, semS0),
                     lambda: wait_scat(fldst1, rows1, semS1))
        lax.cond(fired >= 2 * _G, wait_same, lambda: None)

        lax.cond(k0,
                 lambda: issue(fsrc0, fldst0, rows0, semG0, head),
                 lambda: issue(fsrc1, fldst1, rows1, semG1, head))

        def drain_prev():
            lax.cond(k0,
                     lambda: drain(fsrc1, fldst1, rows1, semG1, semS1),
                     lambda: drain(fsrc0, fldst0, rows0, semG0, semS0))
        lax.cond(fired >= _G, drain_prev, lambda: None)
        return (cnt, fired + _G)

    for jx in range(2):
        job = c + 2 * jx
        base = job * CH

        zs = s * _CPT

        def init_full():
            pltpu.sync_copy(g1_hbm.at[pl.ds(base + zs, _CPT), :],
                            acc.at[pl.ds(zs, _CPT), :])

        def init_last():
            nreal = CH - 15 * _CPT
            pltpu.sync_copy(g1_hbm.at[pl.ds(base + 15 * _CPT, nreal), :],
                            acc.at[pl.ds(15 * _CPT, nreal), :])
        lax.cond(s < 15, init_full, init_last)
        plsc.subcore_barrier()

        def filt_batch(sb, db, st):
            cnt, fired = st

            def filt(i, cnt):
                dstv = db[pl.ds(i * L, L)]
                srcv = sb[pl.ds(i * L, L)]
                loc = dstv - base
                msk = (loc >= 0) & (loc < CH)
                inc = plsc.cumsum(jnp.where(msk, 1, 0).astype(jnp.int32))
                pos = (cnt + inc - 1) & (_CQ - 1)
                plsc.store_scatter(csrc, [pos], srcv, mask=msk)
                plsc.store_scatter(cldst, [pos], jnp.where(msk, loc, 0), mask=msk)
                return cnt + jnp.sum(jnp.where(msk, 1, 0).astype(jnp.int32))

            cnt = lax.fori_loop(0, _B3 // L, filt, cnt)
            return lax.while_loop(lambda st2: st2[0] - st2[1] >= _G,
                                  fire_step, (cnt, fired))

        stage(jnp.int32(0), srcA, dstA, semA)
        stage(jnp.int32(1), srcB, dstB, semB)

        def super_step(sb_i, st):
            b0 = sb_i * 2
            wait_stage(srcA, dstA, semA)
            st = filt_batch(srcA, dstA, st)
            stage(b0 + 2, srcA, dstA, semA)
            wait_stage(srcB, dstB, semB)
            st = filt_batch(srcB, dstB, st)
            stage(b0 + 3, srcB, dstB, semB)
            return st

        cnt, fired = lax.fori_loop(0, _NB3 // 2, super_step,
                                   (jnp.int32(0), jnp.int32(0)))
        wait_stage(srcA, dstA, semA)
        wait_stage(srcB, dstB, semB)

        def pad(i, _):
            q = fired + i * L + lanes
            msk = q >= cnt
            wp = q & (_CQ - 1)
            plsc.store_scatter(csrc, [wp], (q & 255) + s * 512, mask=msk)
            plsc.store_scatter(cldst, [wp], CH + lanes, mask=msk)
            return 0
        lax.fori_loop(0, _G // L, pad, 0)
        cnt2, fired2 = fire_step((cnt, fired))
        k0f = (((fired2 - _G) // _G) & 1) == 0
        lax.cond(k0f,
                 lambda: (drain(fsrc0, fldst0, rows0, semG0, semS0),
                          wait_scat(fldst0, rows0, semS0))[-1],
                 lambda: (drain(fsrc1, fldst1, rows1, semG1, semS1),
                          wait_scat(fldst1, rows1, semS1))[-1])

        def wait_other():
            lax.cond(k0f,
                     lambda: wait_scat(fldst1, rows1, semS1),
                     lambda: wait_scat(fldst0, rows0, semS0))
        lax.cond(fired2 >= 2 * _G, wait_other, lambda: None)

        plsc.subcore_barrier()
        ws = s * (CH // NT)
        pltpu.sync_copy(acc.at[pl.ds(ws, CH // NT), :],
                        out_hbm.at[pl.ds(base + ws, CH // NT), :])
        plsc.subcore_barrier()


_agg64_kernel = functools.partial(
    pl.kernel,
    out_type=_sds((NP, 64)),
    mesh=plsc.VectorSubcoreMesh(**_MESH),
    compiler_params=pltpu.CompilerParams(use_tc_tiling_on_sc=False, needs_layout_passes=False),
    scratch_types=[
        pltpu.VMEM_SHARED((CHP, 64), jnp.float32),
        pltpu.VMEM((_B3,), jnp.int32),
        pltpu.VMEM((_B3,), jnp.int32),
        pltpu.VMEM((_B3,), jnp.int32),
        pltpu.VMEM((_B3,), jnp.int32),
        pltpu.VMEM((_CQ,), jnp.int32),
        pltpu.VMEM((_CQ,), jnp.int32),
        pltpu.VMEM((_G,), jnp.int32),
        pltpu.VMEM((_G,), jnp.int32),
        pltpu.VMEM((_G,), jnp.int32),
        pltpu.VMEM((_G,), jnp.int32),
        pltpu.VMEM((_G, 64), jnp.float32),
        pltpu.VMEM((_G, 64), jnp.float32),
        pltpu.SemaphoreType.DMA,
        pltpu.SemaphoreType.DMA,
        pltpu.SemaphoreType.DMA,
        pltpu.SemaphoreType.DMA,
        pltpu.SemaphoreType.DMA,
        pltpu.SemaphoreType.DMA,
    ],
)(_agg64_body)


_BN = 4096


def _tc2_body(ag_ref, dinv_ref, w1_ref, b1_ref, g1_ref):
    dinv = dinv_ref[...]
    z = dinv[:, None] * (ag_ref[0, :, 0:2] + ag_ref[1, :, 0:2])
    h1 = z[:, 0:1] * w1_ref[0, :][None, :] + z[:, 1:2] * w1_ref[1, :][None, :]
    h1 = jnp.maximum(h1 + b1_ref[...][None, :], 0.0)
    g1_ref[...] = h1 * dinv[:, None]


def _tc2(agg0, dinv, W1, b1):
    return pl.pallas_call(
        _tc2_body,
        grid=(NP // _BN,),
        in_specs=[
            pl.BlockSpec((NC, _BN, 16), lambda i: (0, i, 0)),
            pl.BlockSpec((_BN,), lambda i: (i,)),
            pl.BlockSpec((2, 64), lambda i: (0, 0)),
            pl.BlockSpec((64,), lambda i: (0,)),
        ],
        out_specs=pl.BlockSpec((_BN, 64), lambda i: (i, 0)),
        out_shape=_sds((NP, 64)),
    )(agg0, dinv, W1, b1)


def _tc3_body(ag_ref, dinv_ref, w2_ref, b2_ref, wfc_ref, bfc_ref, o_ref):
    dinv = dinv_ref[...]
    z = dinv[:, None] * ag_ref[...]
    h2 = jnp.dot(z, w2_ref[...], preferred_element_type=jnp.float32)
    h2 = jnp.maximum(h2 + b2_ref[...][None, :], 0.0)
    o = jnp.dot(h2, wfc_ref[...], preferred_element_type=jnp.float32)
    o = o[:, 0] + bfc_ref[0]
    o_ref[...] = 1.0 / (1.0 + jnp.exp(-o))


def _tc3(agg1, dinv, W2, b2, Wfc, bfc):
    return pl.pallas_call(
        _tc3_body,
        grid=(NP // _BN,),
        in_specs=[
            pl.BlockSpec((_BN, 64), lambda i: (i, 0)),
            pl.BlockSpec((_BN,), lambda i: (i,)),
            pl.BlockSpec((64, 64), lambda i: (0, 0)),
            pl.BlockSpec((64,), lambda i: (0,)),
            pl.BlockSpec((64, 1), lambda i: (0, 0)),
            pl.BlockSpec((1,), lambda i: (0,)),
        ],
        out_specs=pl.BlockSpec((_BN,), lambda i: (i,)),
        out_shape=_sds((NP,)),
    )(agg1, dinv, W2, b2, Wfc, bfc)


def kernel(x, edge_index, W1, b1, W2, b2, Wfc, bfc):
    ei = edge_index.astype(jnp.int32)
    src = ei[0]
    dst = ei[1]
    x_pad = jnp.pad(x, ((0, NP - N), (0, 0)))
    x0 = x_pad[:, 0]
    x1 = x_pad[:, 1]

    g0, dinv = _deg_kernel(dst, x0, x1)
    agg0 = _agg2_kernel(src, dst, g0)
    g1 = _tc2(agg0, dinv, W1, b1)
    agg1 = _agg64_kernel(src, dst, g1)
    o = _tc3(agg1, dinv, W2, b2, Wfc, bfc)
    return o[:N]
